# baseline (speedup 1.0000x reference)
import functools

import jax
import jax.numpy as jnp
from jax import lax
from jax.experimental import pallas as pl
from jax.experimental.pallas import tpu as pltpu
from jax.experimental.pallas import tpu_sc as plsc

NUM_EMB = 100000
D = 128
B = 16384
H = 50
HP = 56
FLAT = B * HP
NC = 2
NS = 16
NW = NC * NS
PER_W = FLAT // NW
CHUNK = 128
CHUNKS_PER_W = PER_W // CHUNK
NBUF = 2
GROUPS = CHUNKS_PER_W // NBUF


def _emb_body(x_hbm, real_hbm, imag_hbm, real_out, imag_out,
              idx_v, rbuf, ibuf, rgsem, igsem, rwsem, iwsem):
    wid = lax.axis_index("s") * NC + lax.axis_index("c")
    base_chunk = wid * CHUNKS_PER_W
    pltpu.sync_copy(x_hbm.at[pl.ds(base_chunk, CHUNKS_PER_W)], idx_v)

    def gather(j, b):
        pltpu.async_copy(real_hbm.at[idx_v.at[j]], rbuf.at[b], rgsem.at[b])
        pltpu.async_copy(imag_hbm.at[idx_v.at[j]], ibuf.at[b], igsem.at[b])

    for b in range(NBUF):
        gather(b, b)

    def body(g, carry):
        base = g * NBUF
        for b in range(NBUF):
            j = base + b
            row0 = (base_chunk + j) * CHUNK
            pltpu.make_async_copy(real_hbm.at[idx_v.at[j]], rbuf.at[b],
                                  rgsem.at[b]).wait()
            pltpu.async_copy(rbuf.at[b], real_out.at[pl.ds(row0, CHUNK)],
                             rwsem.at[b])
            pltpu.make_async_copy(imag_hbm.at[idx_v.at[j]], ibuf.at[b],
                                  igsem.at[b]).wait()
            pltpu.async_copy(ibuf.at[b], imag_out.at[pl.ds(row0, CHUNK)],
                             iwsem.at[b])

        @pl.when(g < GROUPS - 1)
        def _():
            for b in range(NBUF):
                j = base + NBUF + b
                row0 = (base_chunk + base - NBUF + b) * CHUNK
                pltpu.make_async_copy(rbuf.at[b],
                                      real_out.at[pl.ds(row0, CHUNK)],
                                      rwsem.at[b]).wait()
                pltpu.make_async_copy(ibuf.at[b],
                                      imag_out.at[pl.ds(row0, CHUNK)],
                                      iwsem.at[b]).wait()
                gather(j, b)

        return carry

    lax.fori_loop(0, GROUPS, body, 0)

    last = GROUPS - 1
    for b in range(NBUF):
        row0 = (base_chunk + last * NBUF + b) * CHUNK
        pltpu.make_async_copy(rbuf.at[b], real_out.at[pl.ds(row0, CHUNK)],
                              rwsem.at[b]).wait()
        pltpu.make_async_copy(ibuf.at[b], imag_out.at[pl.ds(row0, CHUNK)],
                              iwsem.at[b]).wait()


@jax.jit
def _run(x2d, real_w, imag_w):
    mesh = plsc.VectorSubcoreMesh(core_axis_name="c", subcore_axis_name="s")
    f = functools.partial(
        pl.kernel,
        out_type=[
            jax.ShapeDtypeStruct((FLAT, D), jnp.float32),
            jax.ShapeDtypeStruct((FLAT, D), jnp.float32),
        ],
        mesh=mesh,
        scratch_types=[
            pltpu.VMEM((CHUNKS_PER_W, CHUNK), jnp.int32),
            pltpu.VMEM((NBUF, CHUNK, D), jnp.float32),
            pltpu.VMEM((NBUF, CHUNK, D), jnp.float32),
            pltpu.SemaphoreType.DMA((NBUF,)),
            pltpu.SemaphoreType.DMA((NBUF,)),
            pltpu.SemaphoreType.DMA((NBUF,)),
            pltpu.SemaphoreType.DMA((NBUF,)),
        ],
    )(_emb_body)
    return f(x2d, real_w, imag_w)


def kernel(x, real_w, imag_w):
    xp = jnp.pad(x.astype(jnp.int32), ((0, 0), (0, HP - H)))
    x2d = xp.reshape(FLAT // CHUNK, CHUNK)
    real_flat, imag_flat = _run(x2d, real_w, imag_w)
    real = real_flat.reshape(B, HP, D)[:, :H, :]
    imag = imag_flat.reshape(B, HP, D)[:, :H, :]
    return (real, imag)


# speedup vs baseline: 4.5385x; 4.5385x over previous
import functools

import jax
import jax.numpy as jnp
from jax import lax
from jax.experimental import pallas as pl
from jax.experimental.pallas import tpu as pltpu
from jax.experimental.pallas import tpu_sc as plsc

NUM_EMB = 100000
D = 128
B = 16384
H = 50
HP = 56
FLAT = B * HP
NC = 2
NS = 16
NW = NC * NS
PER_W = FLAT // NW
CHUNK = 128
CHUNKS_PER_W = PER_W // CHUNK
NBUF = 2
GROUPS = CHUNKS_PER_W // NBUF


def _emb_body(x_hbm, real_hbm, imag_hbm, real_out, imag_out,
              idx_v, rbuf, ibuf, rgsem, igsem, rwsem, iwsem):
    wid = lax.axis_index("s") * NC + lax.axis_index("c")
    base_chunk = wid * CHUNKS_PER_W
    pltpu.sync_copy(x_hbm.at[pl.ds(base_chunk, CHUNKS_PER_W)], idx_v)

    def gather(j, b):
        pltpu.async_copy(real_hbm.at[idx_v.at[j]], rbuf.at[b], rgsem.at[b])
        pltpu.async_copy(imag_hbm.at[idx_v.at[j]], ibuf.at[b], igsem.at[b])

    for b in range(NBUF):
        gather(b, b)

    def body(g, carry):
        base = g * NBUF
        for b in range(NBUF):
            j = base + b
            row0 = (base_chunk + j) * CHUNK
            pltpu.make_async_copy(real_hbm.at[idx_v.at[j]], rbuf.at[b],
                                  rgsem.at[b]).wait()
            pltpu.async_copy(rbuf.at[b], real_out.at[pl.ds(row0, CHUNK)],
                             rwsem.at[b])
            pltpu.make_async_copy(imag_hbm.at[idx_v.at[j]], ibuf.at[b],
                                  igsem.at[b]).wait()
            pltpu.async_copy(ibuf.at[b], imag_out.at[pl.ds(row0, CHUNK)],
                             iwsem.at[b])

        @pl.when(g < GROUPS - 1)
        def _():
            for b in range(NBUF):
                j = base + NBUF + b
                row0 = (base_chunk + base - NBUF + b) * CHUNK
                pltpu.make_async_copy(rbuf.at[b],
                                      real_out.at[pl.ds(row0, CHUNK)],
                                      rwsem.at[b]).wait()
                pltpu.make_async_copy(ibuf.at[b],
                                      imag_out.at[pl.ds(row0, CHUNK)],
                                      iwsem.at[b]).wait()
                gather(j, b)

        return carry

    lax.fori_loop(0, GROUPS, body, 0)

    last = GROUPS - 1
    for b in range(NBUF):
        row0 = (base_chunk + last * NBUF + b) * CHUNK
        pltpu.make_async_copy(rbuf.at[b], real_out.at[pl.ds(row0, CHUNK)],
                              rwsem.at[b]).wait()
        pltpu.make_async_copy(ibuf.at[b], imag_out.at[pl.ds(row0, CHUNK)],
                              iwsem.at[b]).wait()


@jax.jit
def _run(x2d, real_w, imag_w):
    mesh = plsc.VectorSubcoreMesh(core_axis_name="c", subcore_axis_name="s")
    f = functools.partial(
        pl.kernel,
        out_type=[
            jax.ShapeDtypeStruct((FLAT, D), jnp.float32),
            jax.ShapeDtypeStruct((FLAT, D), jnp.float32),
        ],
        mesh=mesh,
        scratch_types=[
            pltpu.VMEM((CHUNKS_PER_W, CHUNK), jnp.int32),
            pltpu.VMEM((NBUF, CHUNK, D), jnp.float32),
            pltpu.VMEM((NBUF, CHUNK, D), jnp.float32),
            pltpu.SemaphoreType.DMA((NBUF,)),
            pltpu.SemaphoreType.DMA((NBUF,)),
            pltpu.SemaphoreType.DMA((NBUF,)),
            pltpu.SemaphoreType.DMA((NBUF,)),
        ],
    )(_emb_body)
    return f(x2d, real_w, imag_w)


def kernel(x, real_w, imag_w):
    xi = x.astype(jnp.int32)
    xp = jnp.concatenate([xi, xi[:, : HP - H]], axis=1)
    x2d = xp.reshape(FLAT // CHUNK, CHUNK)
    real_flat, imag_flat = _run(x2d, real_w, imag_w)
    real = real_flat.reshape(B, HP, D)[:, :H, :]
    imag = imag_flat.reshape(B, HP, D)[:, :H, :]
    return (real, imag)


# speedup vs baseline: 5.2408x; 1.1547x over previous
import functools

import jax
import jax.numpy as jnp
from jax import lax
from jax.experimental import pallas as pl
from jax.experimental.pallas import tpu as pltpu
from jax.experimental.pallas import tpu_sc as plsc

NUM_EMB = 100000
D = 128
B = 16384
H = 50
HP = 56
NC = 2
NS = 16
NW = NC * NS
SAMP_PER_W = B // NW
PAIRS_PER_W = SAMP_PER_W // 2
IDX_PER_W = SAMP_PER_W * HP
NBUF = 2
GROUPS = PAIRS_PER_W // NBUF


def _emb_body(x_hbm, real_hbm, imag_hbm, real_out, imag_out,
              idx_v, rbuf, ibuf, rgsem, igsem, rwsem, iwsem):
    wid = lax.axis_index("s") * NC + lax.axis_index("c")
    base_s = wid * SAMP_PER_W
    pltpu.sync_copy(x_hbm.at[pl.ds(wid * IDX_PER_W, IDX_PER_W)], idx_v)

    def gather(p, b):
        sl = idx_v.at[pl.ds(p * (2 * HP), 2 * HP)]
        pltpu.async_copy(real_hbm.at[sl], rbuf.at[b], rgsem.at[b])
        pltpu.async_copy(imag_hbm.at[sl], ibuf.at[b], igsem.at[b])

    def write(p, b, buf, out, wsem):
        s0 = base_s + 2 * p
        pltpu.async_copy(buf.at[b, pl.ds(0, H)], out.at[s0], wsem.at[b])
        pltpu.async_copy(buf.at[b, pl.ds(HP, H)], out.at[s0 + 1], wsem.at[b])

    def wait_writes(b, buf, out, wsem):
        pltpu.make_async_copy(buf.at[b, pl.ds(0, H)], out.at[base_s],
                              wsem.at[b]).wait()
        pltpu.make_async_copy(buf.at[b, pl.ds(HP, H)], out.at[base_s],
                              wsem.at[b]).wait()

    for b in range(NBUF):
        gather(b, b)

    def body(g, carry):
        base = g * NBUF
        for b in range(NBUF):
            p = base + b
            pltpu.make_async_copy(real_hbm.at[idx_v.at[pl.ds(0, 2 * HP)]],
                                  rbuf.at[b], rgsem.at[b]).wait()
            write(p, b, rbuf, real_out, rwsem)
            pltpu.make_async_copy(imag_hbm.at[idx_v.at[pl.ds(0, 2 * HP)]],
                                  ibuf.at[b], igsem.at[b]).wait()
            write(p, b, ibuf, imag_out, iwsem)

        @pl.when(g < GROUPS - 1)
        def _():
            for b in range(NBUF):
                p = base + NBUF + b
                wait_writes(b, rbuf, real_out, rwsem)
                wait_writes(b, ibuf, imag_out, iwsem)
                gather(p, b)

        return carry

    lax.fori_loop(0, GROUPS, body, 0)

    for b in range(NBUF):
        wait_writes(b, rbuf, real_out, rwsem)
        wait_writes(b, ibuf, imag_out, iwsem)


@jax.jit
def _run(x1d, real_w, imag_w):
    mesh = plsc.VectorSubcoreMesh(core_axis_name="c", subcore_axis_name="s")
    f = functools.partial(
        pl.kernel,
        out_type=[
            jax.ShapeDtypeStruct((B, H, D), jnp.float32),
            jax.ShapeDtypeStruct((B, H, D), jnp.float32),
        ],
        mesh=mesh,
        scratch_types=[
            pltpu.VMEM((IDX_PER_W,), jnp.int32),
            pltpu.VMEM((NBUF, 2 * HP, D), jnp.float32),
            pltpu.VMEM((NBUF, 2 * HP, D), jnp.float32),
            pltpu.SemaphoreType.DMA((NBUF,)),
            pltpu.SemaphoreType.DMA((NBUF,)),
            pltpu.SemaphoreType.DMA((NBUF,)),
            pltpu.SemaphoreType.DMA((NBUF,)),
        ],
    )(_emb_body)
    return f(x1d, real_w, imag_w)


def kernel(x, real_w, imag_w):
    xi = x.astype(jnp.int32)
    xp = jnp.concatenate([xi, xi[:, : HP - H]], axis=1)
    real3, imag3 = _run(xp.reshape(B * HP), real_w, imag_w)
    return (real3, imag3)


# speedup vs baseline: 5.2510x; 1.0020x over previous
import functools

import jax
import jax.numpy as jnp
from jax import lax
from jax.experimental import pallas as pl
from jax.experimental.pallas import tpu as pltpu
from jax.experimental.pallas import tpu_sc as plsc

NUM_EMB = 100000
D = 128
B = 16384
H = 50
HP = 56
NC = 2
NS = 16
NW = NC * NS
SAMP_PER_W = B // NW
PAIRS_PER_W = SAMP_PER_W // 2
IDX_PER_W = SAMP_PER_W * HP
NBUF = 2
GROUPS = PAIRS_PER_W // NBUF


def _emb_body(x_hbm, real_hbm, imag_hbm, real_out, imag_out,
              idx_v, rbuf, ibuf, rgsem, igsem, rwsem, iwsem):
    wid = lax.axis_index("s") * NC + lax.axis_index("c")
    base_s = wid * SAMP_PER_W
    pltpu.sync_copy(x_hbm.at[pl.ds(wid * IDX_PER_W, IDX_PER_W)], idx_v)

    def gather(p, b):
        sl = idx_v.at[pl.ds(p * (2 * HP), 2 * HP)]
        pltpu.async_copy(real_hbm.at[sl], rbuf.at[b], rgsem.at[b])
        pltpu.async_copy(imag_hbm.at[sl], ibuf.at[b], igsem.at[b])

    def write(p, b, buf, out, wsem):
        s0 = base_s + 2 * p
        pltpu.async_copy(buf.at[b, pl.ds(0, H)], out.at[s0], wsem.at[b])
        pltpu.async_copy(buf.at[b, pl.ds(HP, H)], out.at[s0 + 1], wsem.at[b])

    def wait_writes(b, buf, out, wsem):
        pltpu.make_async_copy(buf.at[b, pl.ds(0, H)], out.at[base_s],
                              wsem.at[b]).wait()
        pltpu.make_async_copy(buf.at[b, pl.ds(HP, H)], out.at[base_s],
                              wsem.at[b]).wait()

    for b in range(NBUF):
        gather(b, b)

    def body(g, carry):
        base = g * NBUF
        for b in range(NBUF):
            p = base + b
            pltpu.make_async_copy(real_hbm.at[idx_v.at[pl.ds(0, 2 * HP)]],
                                  rbuf.at[b], rgsem.at[b]).wait()
            write(p, b, rbuf, real_out, rwsem)
            pltpu.make_async_copy(imag_hbm.at[idx_v.at[pl.ds(0, 2 * HP)]],
                                  ibuf.at[b], igsem.at[b]).wait()
            write(p, b, ibuf, imag_out, iwsem)

        @pl.when(g < GROUPS - 1)
        def _():
            for b in range(NBUF):
                p = base + NBUF + b
                wait_writes(b, rbuf, real_out, rwsem)
                wait_writes(b, ibuf, imag_out, iwsem)
                gather(p, b)

        return carry

    lax.fori_loop(0, GROUPS, body, 0)

    for b in range(NBUF):
        wait_writes(b, rbuf, real_out, rwsem)
        wait_writes(b, ibuf, imag_out, iwsem)


@jax.jit
def _run(x1d, real_w, imag_w):
    mesh = plsc.VectorSubcoreMesh(core_axis_name="c", subcore_axis_name="s")
    f = functools.partial(
        pl.kernel,
        out_type=[
            jax.ShapeDtypeStruct((B, H, D), jnp.float32),
            jax.ShapeDtypeStruct((B, H, D), jnp.float32),
        ],
        mesh=mesh,
        compiler_params=pltpu.CompilerParams(use_tc_tiling_on_sc=True),
        scratch_types=[
            pltpu.VMEM((IDX_PER_W,), jnp.int32),
            pltpu.VMEM((NBUF, 2 * HP, D), jnp.float32),
            pltpu.VMEM((NBUF, 2 * HP, D), jnp.float32),
            pltpu.SemaphoreType.DMA((NBUF,)),
            pltpu.SemaphoreType.DMA((NBUF,)),
            pltpu.SemaphoreType.DMA((NBUF,)),
            pltpu.SemaphoreType.DMA((NBUF,)),
        ],
    )(_emb_body)
    return f(x1d, real_w, imag_w)


def kernel(x, real_w, imag_w):
    xi = x.astype(jnp.int32)
    xp = jnp.concatenate([xi, xi[:, : HP - H]], axis=1)
    real3, imag3 = _run(xp.reshape(B * HP), real_w, imag_w)
    return (real3, imag3)
